# baseline (speedup 1.0000x reference)
import functools

import jax
import jax.numpy as jnp
from jax import lax
from jax.experimental import pallas as pl
from jax.experimental.pallas import tpu as pltpu
from jax.experimental.pallas import tpu_sc as plsc

T = 2048
D = 1024
K = 15
KS = 16
PW = 16
RH = 64
BT = 256
NEG = -1e30
HP = jax.lax.Precision.HIGHEST


def _dotT(a, b):
    return lax.dot_general(a.astype(jnp.bfloat16), b.astype(jnp.bfloat16),
                           (((1,), (1,)), ((), ())),
                           preferred_element_type=jnp.float32)


def _dot(a, b):
    return lax.dot_general(a.astype(jnp.bfloat16), b.astype(jnp.bfloat16),
                           (((1,), (0,)), ((), ())),
                           preferred_element_type=jnp.float32)


def _proj_body(x_ref, wi_ref, wp_ref, c_ref, s_ref, i_ref, p_ref, pn_ref, ni_ref, np_ref):
    half = D // 2
    x = x_ref[...]
    I0 = _dotT(x, wi_ref[...])
    P0 = _dotT(x, wp_ref[...])
    c = c_ref[...]
    s = s_ref[...]

    def rope2(A):
        a1 = A[:, :half]
        a2 = A[:, half:]
        return a1 * c - a2 * s, a1 * s + a2 * c

    i1, i2 = rope2(I0)
    p1, p2 = rope2(P0)
    i_ref[:, :half] = i1
    i_ref[:, half:] = i2
    p_ref[:, :half] = p1
    p_ref[:, half:] = p2
    nI = jnp.maximum(jnp.sqrt(jnp.sum(i1 * i1 + i2 * i2, axis=1, keepdims=True)), 1e-12)
    nP = jnp.maximum(jnp.sqrt(jnp.sum(p1 * p1 + p2 * p2, axis=1, keepdims=True)), 1e-12)
    ni_ref[...] = nI
    np_ref[...] = nP
    inv = 1.0 / nP
    pn_ref[:, :half] = p1 * inv
    pn_ref[:, half:] = p2 * inv


def _k1(x2, Wi, Wp, cosb, sinb):
    return pl.pallas_call(
        _proj_body,
        grid=(T // BT,),
        in_specs=[
            pl.BlockSpec((BT, D), lambda i: (i, 0)),
            pl.BlockSpec((D, D), lambda i: (0, 0)),
            pl.BlockSpec((D, D), lambda i: (0, 0)),
            pl.BlockSpec((BT, D // 2), lambda i: (i, 0)),
            pl.BlockSpec((BT, D // 2), lambda i: (i, 0)),
        ],
        out_specs=[
            pl.BlockSpec((BT, D), lambda i: (i, 0)),
            pl.BlockSpec((BT, D), lambda i: (i, 0)),
            pl.BlockSpec((BT, D), lambda i: (i, 0)),
            pl.BlockSpec((BT, 1), lambda i: (i, 0)),
            pl.BlockSpec((BT, 1), lambda i: (i, 0)),
        ],
        out_shape=[
            jax.ShapeDtypeStruct((T, D), jnp.float32),
            jax.ShapeDtypeStruct((T, D), jnp.float32),
            jax.ShapeDtypeStruct((T, D), jnp.float32),
            jax.ShapeDtypeStruct((T, 1), jnp.float32),
            jax.ShapeDtypeStruct((T, 1), jnp.float32),
        ],
    )(x2, Wi, Wp, cosb, sinb)


def _sg_body(i_ref, p_ref, pnt_ref, pns_ref, s_ref, g_ref):
    ti = pl.program_id(0)
    si = pl.program_id(1)
    scale = D ** -0.5
    S = _dotT(i_ref[...], p_ref[...]) * scale
    row = lax.broadcasted_iota(jnp.int32, (BT, BT), 0) + ti * BT
    col = lax.broadcasted_iota(jnp.int32, (BT, BT), 1) + si * BT
    s_ref[...] = jnp.where(row >= col, S, NEG)
    g_ref[...] = _dotT(pnt_ref[...], pns_ref[...])


def _k2(I, P, Pn):
    return pl.pallas_call(
        _sg_body,
        grid=(T // BT, T // BT),
        in_specs=[
            pl.BlockSpec((BT, D), lambda i, j: (i, 0)),
            pl.BlockSpec((BT, D), lambda i, j: (j, 0)),
            pl.BlockSpec((BT, D), lambda i, j: (i, 0)),
            pl.BlockSpec((BT, D), lambda i, j: (j, 0)),
        ],
        out_specs=[
            pl.BlockSpec((BT, BT), lambda i, j: (i, j)),
            pl.BlockSpec((BT, BT), lambda i, j: (i, j)),
        ],
        out_shape=[
            jax.ShapeDtypeStruct((T, T), jnp.float32),
            jax.ShapeDtypeStruct((T, T), jnp.float32),
        ],
    )(I, P, Pn, Pn)


def _topk_body(s_ref, tv_ref, ti_ref, fl_ref):
    Sw = s_ref[...]
    lane = lax.broadcasted_iota(jnp.int32, (BT, T), 1)
    vals = []
    idxs = []
    for _ in range(K):
        m = jnp.max(Sw, axis=1, keepdims=True)
        am = jnp.min(jnp.where(Sw >= m, lane, T), axis=1, keepdims=True)
        vals.append(m)
        idxs.append(am)
        Sw = jnp.where(lane == am, NEG, Sw)
    tv = jnp.concatenate(vals + [jnp.full((BT, 1), NEG, jnp.float32)], axis=1)
    ti = jnp.concatenate(idxs + [jnp.zeros((BT, 1), jnp.int32)], axis=1)
    tv_ref[...] = tv
    ti_ref[...] = ti
    for k in range(K):
        fl_ref[:, k * KS:(k + 1) * KS] = idxs[k] * T + ti


def _k3(S):
    return pl.pallas_call(
        _topk_body,
        grid=(T // BT,),
        in_specs=[pl.BlockSpec((BT, T), lambda i: (i, 0))],
        out_specs=[
            pl.BlockSpec((BT, KS), lambda i: (i, 0)),
            pl.BlockSpec((BT, KS), lambda i: (i, 0)),
            pl.BlockSpec((BT, K * KS), lambda i: (i, 0)),
        ],
        out_shape=[
            jax.ShapeDtypeStruct((T, KS), jnp.float32),
            jax.ShapeDtypeStruct((T, KS), jnp.int32),
            jax.ShapeDtypeStruct((T, K * KS), jnp.int32),
        ],
    )(S)


NW = 32
TPW = T // NW


FPW = TPW * K * KS
IPW = TPW * KS
CH = 128


def _sc_body(fl_hbm, idx_hbm, png_hbm, np_hbm, g_out, np_out,
             fl_v, g_v, idx_v, np_v, sem1, sem2):
    wid = lax.axis_index("s") * 2 + lax.axis_index("c")
    pltpu.sync_copy(fl_hbm.at[pl.ds(wid * FPW, FPW)], fl_v)
    pltpu.sync_copy(idx_hbm.at[pl.ds(wid * IPW, IPW)], idx_v)

    def gat_g(i, _):
        pltpu.async_copy(png_hbm.at[fl_v.at[pl.ds(i * CH, CH)]],
                         g_v.at[pl.ds(i * CH, CH)], sem1).wait()
        return 0

    lax.fori_loop(0, FPW // CH, gat_g, 0)

    def gat_n(i, _):
        pltpu.async_copy(np_hbm.at[idx_v.at[pl.ds(i * CH, CH)]],
                         np_v.at[pl.ds(i * CH, CH)], sem2).wait()
        return 0

    lax.fori_loop(0, IPW // CH, gat_n, 0)
    pltpu.sync_copy(g_v, g_out.at[pl.ds(wid * FPW, FPW)])
    pltpu.sync_copy(np_v, np_out.at[pl.ds(wid * IPW, IPW)])


def _k4(flat2, idx16, png_flat, np_flat):
    mesh = plsc.VectorSubcoreMesh(core_axis_name="c", subcore_axis_name="s")
    f = functools.partial(
        pl.kernel,
        mesh=mesh,
        out_type=[
            jax.ShapeDtypeStruct((T * K * KS,), jnp.float32),
            jax.ShapeDtypeStruct((T * KS,), jnp.float32),
        ],
        scratch_types=[
            pltpu.VMEM((FPW,), jnp.int32),
            pltpu.VMEM((FPW,), jnp.float32),
            pltpu.VMEM((IPW,), jnp.int32),
            pltpu.VMEM((IPW,), jnp.float32),
            pltpu.SemaphoreType.DMA,
            pltpu.SemaphoreType.DMA,
        ],
    )(_sc_body)
    return f(flat2, idx16, png_flat, np_flat)


def _feat_body(tv_ref, ti_ref, ni_ref, nps_ref, g_ref, feat_ref, dl_ref, gm_ref):
    pid = pl.program_id(0)
    tcol = lax.broadcasted_iota(jnp.int32, (BT, 1), 0) + pid * BT
    lane = lax.broadcasted_iota(jnp.int32, (BT, KS), 1)
    keep = (lane <= tcol) & (lane < K)
    kf = keep.astype(jnp.float32)
    tv = tv_ref[...]
    ti = ti_ref[...]
    nI = ni_ref[...]
    nps = nps_ref[...]
    inv_scale = float(D) ** 0.5
    feat_ref[...] = jnp.clip(tv * inv_scale / (nI * nps), -1.0, 1.0) * kf
    dl_ref[...] = jnp.maximum((tcol - ti).astype(jnp.float32), 0.0) * (1.0 / T) * kf
    for k in range(K):
        gk = jnp.clip(g_ref[:, k * KS:(k + 1) * KS], -1.0, 1.0)
        gm_ref[:, k * KS:(k + 1) * KS] = gk * kf[:, k:k + 1] * kf


def _k5(tv, ti, nI, nps, G):
    return pl.pallas_call(
        _feat_body,
        grid=(T // BT,),
        in_specs=[
            pl.BlockSpec((BT, KS), lambda i: (i, 0)),
            pl.BlockSpec((BT, KS), lambda i: (i, 0)),
            pl.BlockSpec((BT, 1), lambda i: (i, 0)),
            pl.BlockSpec((BT, KS), lambda i: (i, 0)),
            pl.BlockSpec((BT, K * KS), lambda i: (i, 0)),
        ],
        out_specs=[
            pl.BlockSpec((BT, KS), lambda i: (i, 0)),
            pl.BlockSpec((BT, KS), lambda i: (i, 0)),
            pl.BlockSpec((BT, K * KS), lambda i: (i, 0)),
        ],
        out_shape=[
            jax.ShapeDtypeStruct((T, KS), jnp.float32),
            jax.ShapeDtypeStruct((T, KS), jnp.float32),
            jax.ShapeDtypeStruct((T, K * KS), jnp.float32),
        ],
    )(tv, ti, nI, nps, G)


BM = 1024


def _gelu_exact(x):
    return x * 0.5 * (1.0 + lax.erf(x * (2.0 ** -0.5)))


def _mlp_body(x_ref, w1_ref, b1_ref, w2_ref, b2_ref, wh_ref, bh_ref, o_ref):
    h = _dot(x_ref[...], w1_ref[...]) + b1_ref[...]
    h = _gelu_exact(h)
    h = _dot(h, w2_ref[...]) + b2_ref[...]
    h = _gelu_exact(h)
    o = _dot(h, wh_ref[...]) + bh_ref[...]
    o_ref[...] = jnp.concatenate([jnp.tanh(o[:, :2]), o[:, 2:]], axis=1)


def _k6(relp, W1p, b1p, W2p, b2p, Whp, bhp):
    NROW = T * K
    return pl.pallas_call(
        _mlp_body,
        grid=(NROW // BM,),
        in_specs=[
            pl.BlockSpec((BM, 128), lambda i: (i, 0)),
            pl.BlockSpec((128, RH), lambda i: (0, 0)),
            pl.BlockSpec((1, RH), lambda i: (0, 0)),
            pl.BlockSpec((RH, RH), lambda i: (0, 0)),
            pl.BlockSpec((1, RH), lambda i: (0, 0)),
            pl.BlockSpec((RH, 8), lambda i: (0, 0)),
            pl.BlockSpec((1, 8), lambda i: (0, 0)),
        ],
        out_specs=pl.BlockSpec((BM, 8), lambda i: (i, 0)),
        out_shape=jax.ShapeDtypeStruct((NROW, 8), jnp.float32),
    )(relp, W1p, b1p, W2p, b2p, Whp, bhp)


def _coef_body(z0_ref, z1_ref, m_ref, coef_ref):
    pid = pl.program_id(0)
    tcol = lax.broadcasted_iota(jnp.int32, (BT, 1), 0) + pid * BT
    lane = lax.broadcasted_iota(jnp.int32, (BT, KS), 1)
    keep = (lane <= tcol) & (lane < K)
    kf = keep.astype(jnp.float32)
    mm = jnp.where(keep, m_ref[...], NEG)
    mx = jnp.max(mm, axis=1, keepdims=True)
    e = jnp.exp(mm - mx) * kf
    w = e / jnp.sum(e, axis=1, keepdims=True)

    z0 = z0_ref[...]
    z1 = z1_ref[...]
    ix = jnp.clip((z0 + 1.0) * (0.5 * (PW - 1)), 0.0, PW - 1.0)
    iy = jnp.clip((z1 + 1.0) * (0.5 * (PW - 1)), 0.0, PW - 1.0)
    ix0f = jnp.floor(ix)
    iy0f = jnp.floor(iy)
    wx1 = ix - ix0f
    wy1 = iy - iy0f
    wx0 = 1.0 - wx1
    wy0 = 1.0 - wy1
    ix0 = jnp.clip(ix0f.astype(jnp.int32), 0, PW - 1)
    iy0 = jnp.clip(iy0f.astype(jnp.int32), 0, PW - 1)
    ix1 = jnp.clip(ix0f.astype(jnp.int32) + 1, 0, PW - 1)
    iy1 = jnp.clip(iy0f.astype(jnp.int32) + 1, 0, PW - 1)

    lane256 = lax.broadcasted_iota(jnp.int32, (BT, PW * PW), 1)
    coef = jnp.zeros((BT, PW * PW), jnp.float32)
    for k in range(K):
        wk = w[:, k:k + 1]
        for yy, xx, wy, wx in ((iy0, ix0, wy0, wx0), (iy0, ix1, wy0, wx1),
                               (iy1, ix0, wy1, wx0), (iy1, ix1, wy1, wx1)):
            pos = yy[:, k:k + 1] * PW + xx[:, k:k + 1]
            amp = wk * (wy[:, k:k + 1] * wx[:, k:k + 1])
            coef = coef + jnp.where(lane256 == pos, amp, 0.0)
    coef_ref[...] = coef


def _k7(z0p, z1p, mp):
    return pl.pallas_call(
        _coef_body,
        grid=(T // BT,),
        in_specs=[
            pl.BlockSpec((BT, KS), lambda i: (i, 0)),
            pl.BlockSpec((BT, KS), lambda i: (i, 0)),
            pl.BlockSpec((BT, KS), lambda i: (i, 0)),
        ],
        out_specs=pl.BlockSpec((BT, PW * PW), lambda i: (i, 0)),
        out_shape=jax.ShapeDtypeStruct((T, PW * PW), jnp.float32),
    )(z0p, z1p, mp)


def _pal_body(pal_ref, wo_ref, m_ref):
    m_ref[...] = lax.dot_general(pal_ref[...].astype(jnp.bfloat16),
                                 wo_ref[...].astype(jnp.bfloat16),
                                 (((0,), (0,)), ((), ())),
                                 preferred_element_type=jnp.float32)


def _k8(pal2, Wo0):
    return pl.pallas_call(
        _pal_body,
        in_specs=[
            pl.BlockSpec((D, PW * PW), lambda: (0, 0)),
            pl.BlockSpec((D, D), lambda: (0, 0)),
        ],
        out_specs=pl.BlockSpec((PW * PW, D), lambda: (0, 0)),
        out_shape=jax.ShapeDtypeStruct((PW * PW, D), jnp.float32),
    )(pal2, Wo0)


def _out_body(c_ref, m_ref, y_ref):
    y_ref[...] = _dot(c_ref[...], m_ref[...])


def _k9(coef, M):
    return pl.pallas_call(
        _out_body,
        grid=(T // BT,),
        in_specs=[
            pl.BlockSpec((BT, PW * PW), lambda i: (i, 0)),
            pl.BlockSpec((PW * PW, D), lambda i: (0, 0)),
        ],
        out_specs=pl.BlockSpec((BT, D), lambda i: (i, 0)),
        out_shape=jax.ShapeDtypeStruct((T, D), jnp.float32),
    )(coef, M)


def kernel(x, Wi, Wp, palette, W1, b1, W2, b2, Wc, bc, Wm, bm, Wo):
    x2 = x.reshape(T, D)
    half = D // 2
    freqs = 1.0 / (10000.0 ** (jnp.arange(half, dtype=jnp.float32) / half))
    ang = jnp.arange(T, dtype=jnp.float32)[:, None] * freqs[None, :]
    cosb = jnp.cos(ang)
    sinb = jnp.sin(ang)
    I, P, Pn, nI, nP = _k1(x2, Wi, Wp, cosb, sinb)
    S, PnG = _k2(I, P, Pn)
    tv, ti, flat = _k3(S)

    flat2 = flat.reshape(T * K * KS)
    idx1 = ti.reshape(T * KS)
    png_flat = PnG.reshape(T * T)
    np_flat = nP.reshape(T)
    g_flat, nps1 = _k4(flat2, idx1, png_flat, np_flat)
    G = g_flat.reshape(T, K * KS)
    nps = nps1.reshape(T, KS)

    feat, dl, Gm = _k5(tv, ti, nI, nps, G)

    Gr = Gm.reshape(T, K, KS)[:, :, :K]
    rel = jnp.concatenate([Gr, feat[:, :K, None], dl[:, :K, None]], axis=2)
    rel = rel.reshape(T * K, K + 2)
    relp = jnp.pad(rel, ((0, 0), (0, 128 - (K + 2))))

    W1p = jnp.pad(W1.T, ((0, 128 - (K + 2)), (0, 0)))
    b1p = b1.reshape(1, RH)
    W2p = W2.T
    b2p = b2.reshape(1, RH)
    Whp = jnp.pad(jnp.concatenate([Wc.T, Wm.T], axis=1), ((0, 0), (0, 5)))
    bhp = jnp.pad(jnp.concatenate([bc, bm]), (0, 5)).reshape(1, 8)

    o3 = _k6(relp, W1p, b1p, W2p, b2p, Whp, bhp)
    o3r = o3.reshape(T, K, 8)
    z0p = jnp.pad(o3r[:, :, 0], ((0, 0), (0, KS - K)))
    z1p = jnp.pad(o3r[:, :, 1], ((0, 0), (0, KS - K)))
    mp = jnp.pad(o3r[:, :, 2], ((0, 0), (0, KS - K)))

    coef = _k7(z0p, z1p, mp)

    pal2 = palette.reshape(D, PW * PW)
    M = _k8(pal2, Wo[0])
    y = _k9(coef, M)
    return y.reshape(1, T, D)


# speedup vs baseline: 1.1135x; 1.1135x over previous
import functools

import jax
import jax.numpy as jnp
from jax import lax
from jax.experimental import pallas as pl
from jax.experimental.pallas import tpu as pltpu
from jax.experimental.pallas import tpu_sc as plsc

T = 2048
D = 1024
K = 15
KS = 16
PW = 16
RH = 64
BT = 256
NEG = -1e30
HP = jax.lax.Precision.HIGHEST


def _dotT(a, b):
    return lax.dot_general(a.astype(jnp.bfloat16), b.astype(jnp.bfloat16),
                           (((1,), (1,)), ((), ())),
                           preferred_element_type=jnp.float32)


def _dot(a, b):
    return lax.dot_general(a.astype(jnp.bfloat16), b.astype(jnp.bfloat16),
                           (((1,), (0,)), ((), ())),
                           preferred_element_type=jnp.float32)


def _proj_body(x_ref, wi_ref, wp_ref, c_ref, s_ref, i_ref, p_ref, pn_ref, ni_ref, np_ref):
    half = D // 2
    x = x_ref[...]
    I0 = _dotT(x, wi_ref[...])
    P0 = _dotT(x, wp_ref[...])
    c = c_ref[...]
    s = s_ref[...]

    def rope2(A):
        a1 = A[:, :half]
        a2 = A[:, half:]
        return a1 * c - a2 * s, a1 * s + a2 * c

    i1, i2 = rope2(I0)
    p1, p2 = rope2(P0)
    i_ref[:, :half] = i1
    i_ref[:, half:] = i2
    p_ref[:, :half] = p1
    p_ref[:, half:] = p2
    nI = jnp.maximum(jnp.sqrt(jnp.sum(i1 * i1 + i2 * i2, axis=1, keepdims=True)), 1e-12)
    nP = jnp.maximum(jnp.sqrt(jnp.sum(p1 * p1 + p2 * p2, axis=1, keepdims=True)), 1e-12)
    ni_ref[...] = nI
    np_ref[...] = nP
    inv = 1.0 / nP
    pn_ref[:, :half] = p1 * inv
    pn_ref[:, half:] = p2 * inv


def _k1(x2, Wi, Wp, cosb, sinb):
    return pl.pallas_call(
        _proj_body,
        grid=(T // BT,),
        in_specs=[
            pl.BlockSpec((BT, D), lambda i: (i, 0)),
            pl.BlockSpec((D, D), lambda i: (0, 0)),
            pl.BlockSpec((D, D), lambda i: (0, 0)),
            pl.BlockSpec((BT, D // 2), lambda i: (i, 0)),
            pl.BlockSpec((BT, D // 2), lambda i: (i, 0)),
        ],
        out_specs=[
            pl.BlockSpec((BT, D), lambda i: (i, 0)),
            pl.BlockSpec((BT, D), lambda i: (i, 0)),
            pl.BlockSpec((BT, D), lambda i: (i, 0)),
            pl.BlockSpec((BT, 1), lambda i: (i, 0)),
            pl.BlockSpec((BT, 1), lambda i: (i, 0)),
        ],
        out_shape=[
            jax.ShapeDtypeStruct((T, D), jnp.float32),
            jax.ShapeDtypeStruct((T, D), jnp.float32),
            jax.ShapeDtypeStruct((T, D), jnp.float32),
            jax.ShapeDtypeStruct((T, 1), jnp.float32),
            jax.ShapeDtypeStruct((T, 1), jnp.float32),
        ],
    )(x2, Wi, Wp, cosb, sinb)


def _sg_body(i_ref, p_ref, pnt_ref, pns_ref, s_ref, g_ref):
    ti = pl.program_id(0)
    si = pl.program_id(1)
    scale = D ** -0.5
    S = _dotT(i_ref[...], p_ref[...]) * scale
    row = lax.broadcasted_iota(jnp.int32, (BT, BT), 0) + ti * BT
    col = lax.broadcasted_iota(jnp.int32, (BT, BT), 1) + si * BT
    s_ref[...] = jnp.where(row >= col, S, NEG)
    g_ref[...] = _dotT(pnt_ref[...], pns_ref[...])


def _k2(I, P, Pn):
    return pl.pallas_call(
        _sg_body,
        grid=(T // BT, T // BT),
        in_specs=[
            pl.BlockSpec((BT, D), lambda i, j: (i, 0)),
            pl.BlockSpec((BT, D), lambda i, j: (j, 0)),
            pl.BlockSpec((BT, D), lambda i, j: (i, 0)),
            pl.BlockSpec((BT, D), lambda i, j: (j, 0)),
        ],
        out_specs=[
            pl.BlockSpec((BT, BT), lambda i, j: (i, j)),
            pl.BlockSpec((BT, BT), lambda i, j: (i, j)),
        ],
        out_shape=[
            jax.ShapeDtypeStruct((T, T), jnp.float32),
            jax.ShapeDtypeStruct((T, T), jnp.float32),
        ],
    )(I, P, Pn, Pn)


def _topk_body(s_ref, tv_ref, ti_ref, fl_ref):
    Sw = s_ref[...]
    lane = lax.broadcasted_iota(jnp.int32, (BT, T), 1)
    vals = []
    idxs = []
    for _ in range(K):
        m = jnp.max(Sw, axis=1, keepdims=True)
        am = jnp.min(jnp.where(Sw >= m, lane, T), axis=1, keepdims=True)
        vals.append(m)
        idxs.append(am)
        Sw = jnp.where(lane == am, NEG, Sw)
    tv = jnp.concatenate(vals + [jnp.full((BT, 1), NEG, jnp.float32)], axis=1)
    ti = jnp.concatenate(idxs + [jnp.zeros((BT, 1), jnp.int32)], axis=1)
    tv_ref[...] = tv
    ti_ref[...] = ti
    for k in range(K):
        fl_ref[:, k * KS:(k + 1) * KS] = idxs[k] * T + ti


def _k3(S):
    return pl.pallas_call(
        _topk_body,
        grid=(T // BT,),
        in_specs=[pl.BlockSpec((BT, T), lambda i: (i, 0))],
        out_specs=[
            pl.BlockSpec((BT, KS), lambda i: (i, 0)),
            pl.BlockSpec((BT, KS), lambda i: (i, 0)),
            pl.BlockSpec((BT, K * KS), lambda i: (i, 0)),
        ],
        out_shape=[
            jax.ShapeDtypeStruct((T, KS), jnp.float32),
            jax.ShapeDtypeStruct((T, KS), jnp.int32),
            jax.ShapeDtypeStruct((T, K * KS), jnp.int32),
        ],
    )(S)


NW = 32
TPW = T // NW


FPW = TPW * K * KS
IPW = TPW * KS
CH = 128
GRP = 8


def _sc_body(fl_hbm, idx_hbm, png_hbm, np_hbm, g_out, np_out,
             fl_v, g_v, idx_v, np_v, sem1, sem2):
    wid = lax.axis_index("s") * 2 + lax.axis_index("c")
    pltpu.sync_copy(fl_hbm.at[pl.ds(wid * FPW, FPW)], fl_v)
    pltpu.sync_copy(idx_hbm.at[pl.ds(wid * IPW, IPW)], idx_v)

    def gat_g(g, _):
        for b in range(GRP):
            off = (g * GRP + b) * CH
            pltpu.async_copy(png_hbm.at[fl_v.at[pl.ds(off, CH)]],
                             g_v.at[pl.ds(off, CH)], sem1)
        for b in range(GRP):
            off = (g * GRP + b) * CH
            pltpu.make_async_copy(png_hbm.at[fl_v.at[pl.ds(off, CH)]],
                                  g_v.at[pl.ds(off, CH)], sem1).wait()
        return 0

    lax.fori_loop(0, FPW // (CH * GRP), gat_g, 0)

    for b in range(IPW // CH):
        pltpu.async_copy(np_hbm.at[idx_v.at[pl.ds(b * CH, CH)]],
                         np_v.at[pl.ds(b * CH, CH)], sem2)
    for b in range(IPW // CH):
        pltpu.make_async_copy(np_hbm.at[idx_v.at[pl.ds(b * CH, CH)]],
                              np_v.at[pl.ds(b * CH, CH)], sem2).wait()
    pltpu.sync_copy(g_v, g_out.at[pl.ds(wid * FPW, FPW)])
    pltpu.sync_copy(np_v, np_out.at[pl.ds(wid * IPW, IPW)])


def _k4(flat2, idx16, png_flat, np_flat):
    mesh = plsc.VectorSubcoreMesh(core_axis_name="c", subcore_axis_name="s")
    f = functools.partial(
        pl.kernel,
        mesh=mesh,
        out_type=[
            jax.ShapeDtypeStruct((T * K * KS,), jnp.float32),
            jax.ShapeDtypeStruct((T * KS,), jnp.float32),
        ],
        scratch_types=[
            pltpu.VMEM((FPW,), jnp.int32),
            pltpu.VMEM((FPW,), jnp.float32),
            pltpu.VMEM((IPW,), jnp.int32),
            pltpu.VMEM((IPW,), jnp.float32),
            pltpu.SemaphoreType.DMA,
            pltpu.SemaphoreType.DMA,
        ],
    )(_sc_body)
    return f(flat2, idx16, png_flat, np_flat)


def _feat_body(tv_ref, ti_ref, ni_ref, nps_ref, g_ref, feat_ref, dl_ref, gm_ref):
    pid = pl.program_id(0)
    tcol = lax.broadcasted_iota(jnp.int32, (BT, 1), 0) + pid * BT
    lane = lax.broadcasted_iota(jnp.int32, (BT, KS), 1)
    keep = (lane <= tcol) & (lane < K)
    kf = keep.astype(jnp.float32)
    tv = tv_ref[...]
    ti = ti_ref[...]
    nI = ni_ref[...]
    nps = nps_ref[...]
    inv_scale = float(D) ** 0.5
    feat_ref[...] = jnp.clip(tv * inv_scale / (nI * nps), -1.0, 1.0) * kf
    dl_ref[...] = jnp.maximum((tcol - ti).astype(jnp.float32), 0.0) * (1.0 / T) * kf
    for k in range(K):
        gk = jnp.clip(g_ref[:, k * KS:(k + 1) * KS], -1.0, 1.0)
        gm_ref[:, k * KS:(k + 1) * KS] = gk * kf[:, k:k + 1] * kf


def _k5(tv, ti, nI, nps, G):
    return pl.pallas_call(
        _feat_body,
        grid=(T // BT,),
        in_specs=[
            pl.BlockSpec((BT, KS), lambda i: (i, 0)),
            pl.BlockSpec((BT, KS), lambda i: (i, 0)),
            pl.BlockSpec((BT, 1), lambda i: (i, 0)),
            pl.BlockSpec((BT, KS), lambda i: (i, 0)),
            pl.BlockSpec((BT, K * KS), lambda i: (i, 0)),
        ],
        out_specs=[
            pl.BlockSpec((BT, KS), lambda i: (i, 0)),
            pl.BlockSpec((BT, KS), lambda i: (i, 0)),
            pl.BlockSpec((BT, K * KS), lambda i: (i, 0)),
        ],
        out_shape=[
            jax.ShapeDtypeStruct((T, KS), jnp.float32),
            jax.ShapeDtypeStruct((T, KS), jnp.float32),
            jax.ShapeDtypeStruct((T, K * KS), jnp.float32),
        ],
    )(tv, ti, nI, nps, G)


BM = 1024


def _gelu_exact(x):
    return x * 0.5 * (1.0 + lax.erf(x * (2.0 ** -0.5)))


def _mlp_body(x_ref, w1_ref, b1_ref, w2_ref, b2_ref, wh_ref, bh_ref, o_ref):
    h = _dot(x_ref[...], w1_ref[...]) + b1_ref[...]
    h = _gelu_exact(h)
    h = _dot(h, w2_ref[...]) + b2_ref[...]
    h = _gelu_exact(h)
    o = _dot(h, wh_ref[...]) + bh_ref[...]
    o_ref[...] = jnp.concatenate([jnp.tanh(o[:, :2]), o[:, 2:]], axis=1)


def _k6(relp, W1p, b1p, W2p, b2p, Whp, bhp):
    NROW = T * K
    return pl.pallas_call(
        _mlp_body,
        grid=(NROW // BM,),
        in_specs=[
            pl.BlockSpec((BM, 128), lambda i: (i, 0)),
            pl.BlockSpec((128, RH), lambda i: (0, 0)),
            pl.BlockSpec((1, RH), lambda i: (0, 0)),
            pl.BlockSpec((RH, RH), lambda i: (0, 0)),
            pl.BlockSpec((1, RH), lambda i: (0, 0)),
            pl.BlockSpec((RH, 8), lambda i: (0, 0)),
            pl.BlockSpec((1, 8), lambda i: (0, 0)),
        ],
        out_specs=pl.BlockSpec((BM, 8), lambda i: (i, 0)),
        out_shape=jax.ShapeDtypeStruct((NROW, 8), jnp.float32),
    )(relp, W1p, b1p, W2p, b2p, Whp, bhp)


def _coef_body(z0_ref, z1_ref, m_ref, coef_ref):
    pid = pl.program_id(0)
    tcol = lax.broadcasted_iota(jnp.int32, (BT, 1), 0) + pid * BT
    lane = lax.broadcasted_iota(jnp.int32, (BT, KS), 1)
    keep = (lane <= tcol) & (lane < K)
    kf = keep.astype(jnp.float32)
    mm = jnp.where(keep, m_ref[...], NEG)
    mx = jnp.max(mm, axis=1, keepdims=True)
    e = jnp.exp(mm - mx) * kf
    w = e / jnp.sum(e, axis=1, keepdims=True)

    z0 = z0_ref[...]
    z1 = z1_ref[...]
    ix = jnp.clip((z0 + 1.0) * (0.5 * (PW - 1)), 0.0, PW - 1.0)
    iy = jnp.clip((z1 + 1.0) * (0.5 * (PW - 1)), 0.0, PW - 1.0)
    ix0f = jnp.floor(ix)
    iy0f = jnp.floor(iy)
    wx1 = ix - ix0f
    wy1 = iy - iy0f
    wx0 = 1.0 - wx1
    wy0 = 1.0 - wy1
    ix0 = jnp.clip(ix0f.astype(jnp.int32), 0, PW - 1)
    iy0 = jnp.clip(iy0f.astype(jnp.int32), 0, PW - 1)
    ix1 = jnp.clip(ix0f.astype(jnp.int32) + 1, 0, PW - 1)
    iy1 = jnp.clip(iy0f.astype(jnp.int32) + 1, 0, PW - 1)

    lane256 = lax.broadcasted_iota(jnp.int32, (BT, PW * PW), 1)
    coef = jnp.zeros((BT, PW * PW), jnp.float32)
    for k in range(K):
        wk = w[:, k:k + 1]
        for yy, xx, wy, wx in ((iy0, ix0, wy0, wx0), (iy0, ix1, wy0, wx1),
                               (iy1, ix0, wy1, wx0), (iy1, ix1, wy1, wx1)):
            pos = yy[:, k:k + 1] * PW + xx[:, k:k + 1]
            amp = wk * (wy[:, k:k + 1] * wx[:, k:k + 1])
            coef = coef + jnp.where(lane256 == pos, amp, 0.0)
    coef_ref[...] = coef


def _k7(z0p, z1p, mp):
    return pl.pallas_call(
        _coef_body,
        grid=(T // BT,),
        in_specs=[
            pl.BlockSpec((BT, KS), lambda i: (i, 0)),
            pl.BlockSpec((BT, KS), lambda i: (i, 0)),
            pl.BlockSpec((BT, KS), lambda i: (i, 0)),
        ],
        out_specs=pl.BlockSpec((BT, PW * PW), lambda i: (i, 0)),
        out_shape=jax.ShapeDtypeStruct((T, PW * PW), jnp.float32),
    )(z0p, z1p, mp)


def _pal_body(pal_ref, wo_ref, m_ref):
    m_ref[...] = lax.dot_general(pal_ref[...].astype(jnp.bfloat16),
                                 wo_ref[...].astype(jnp.bfloat16),
                                 (((0,), (0,)), ((), ())),
                                 preferred_element_type=jnp.float32)


def _k8(pal2, Wo0):
    return pl.pallas_call(
        _pal_body,
        in_specs=[
            pl.BlockSpec((D, PW * PW), lambda: (0, 0)),
            pl.BlockSpec((D, D), lambda: (0, 0)),
        ],
        out_specs=pl.BlockSpec((PW * PW, D), lambda: (0, 0)),
        out_shape=jax.ShapeDtypeStruct((PW * PW, D), jnp.float32),
    )(pal2, Wo0)


def _out_body(c_ref, m_ref, y_ref):
    y_ref[...] = _dot(c_ref[...], m_ref[...])


def _k9(coef, M):
    return pl.pallas_call(
        _out_body,
        grid=(T // BT,),
        in_specs=[
            pl.BlockSpec((BT, PW * PW), lambda i: (i, 0)),
            pl.BlockSpec((PW * PW, D), lambda i: (0, 0)),
        ],
        out_specs=pl.BlockSpec((BT, D), lambda i: (i, 0)),
        out_shape=jax.ShapeDtypeStruct((T, D), jnp.float32),
    )(coef, M)


def kernel(x, Wi, Wp, palette, W1, b1, W2, b2, Wc, bc, Wm, bm, Wo):
    x2 = x.reshape(T, D)
    half = D // 2
    freqs = 1.0 / (10000.0 ** (jnp.arange(half, dtype=jnp.float32) / half))
    ang = jnp.arange(T, dtype=jnp.float32)[:, None] * freqs[None, :]
    cosb = jnp.cos(ang)
    sinb = jnp.sin(ang)
    I, P, Pn, nI, nP = _k1(x2, Wi, Wp, cosb, sinb)
    S, PnG = _k2(I, P, Pn)
    tv, ti, flat = _k3(S)

    flat2 = flat.reshape(T * K * KS)
    idx1 = ti.reshape(T * KS)
    png_flat = PnG.reshape(T * T)
    np_flat = nP.reshape(T)
    g_flat, nps1 = _k4(flat2, idx1, png_flat, np_flat)
    G = g_flat.reshape(T, K * KS)
    nps = nps1.reshape(T, KS)

    feat, dl, Gm = _k5(tv, ti, nI, nps, G)

    Gr = Gm.reshape(T, K, KS)[:, :, :K]
    rel = jnp.concatenate([Gr, feat[:, :K, None], dl[:, :K, None]], axis=2)
    rel = rel.reshape(T * K, K + 2)
    relp = jnp.pad(rel, ((0, 0), (0, 128 - (K + 2))))

    W1p = jnp.pad(W1.T, ((0, 128 - (K + 2)), (0, 0)))
    b1p = b1.reshape(1, RH)
    W2p = W2.T
    b2p = b2.reshape(1, RH)
    Whp = jnp.pad(jnp.concatenate([Wc.T, Wm.T], axis=1), ((0, 0), (0, 5)))
    bhp = jnp.pad(jnp.concatenate([bc, bm]), (0, 5)).reshape(1, 8)

    o3 = _k6(relp, W1p, b1p, W2p, b2p, Whp, bhp)
    o3r = o3.reshape(T, K, 8)
    z0p = jnp.pad(o3r[:, :, 0], ((0, 0), (0, KS - K)))
    z1p = jnp.pad(o3r[:, :, 1], ((0, 0), (0, KS - K)))
    mp = jnp.pad(o3r[:, :, 2], ((0, 0), (0, KS - K)))

    coef = _k7(z0p, z1p, mp)

    pal2 = palette.reshape(D, PW * PW)
    M = _k8(pal2, Wo[0])
    y = _k9(coef, M)
    return y.reshape(1, T, D)


# speedup vs baseline: 1.3573x; 1.2189x over previous
import functools

import jax
import jax.numpy as jnp
from jax import lax
from jax.experimental import pallas as pl
from jax.experimental.pallas import tpu as pltpu
from jax.experimental.pallas import tpu_sc as plsc

T = 2048
D = 1024
K = 15
KS = 16
PW = 16
RH = 64
BT = 256
NEG = -1e30
HP = jax.lax.Precision.HIGHEST


def _dotT(a, b):
    return lax.dot_general(a.astype(jnp.bfloat16), b.astype(jnp.bfloat16),
                           (((1,), (1,)), ((), ())),
                           preferred_element_type=jnp.float32)


def _dot(a, b):
    return lax.dot_general(a.astype(jnp.bfloat16), b.astype(jnp.bfloat16),
                           (((1,), (0,)), ((), ())),
                           preferred_element_type=jnp.float32)


def _proj_body(x_ref, wi_ref, wp_ref, c_ref, s_ref, i_ref, p_ref, pn_ref, ni_ref, np_ref):
    half = D // 2
    x = x_ref[...]
    I0 = _dotT(x, wi_ref[...])
    P0 = _dotT(x, wp_ref[...])
    c = c_ref[...]
    s = s_ref[...]

    def rope2(A):
        a1 = A[:, :half]
        a2 = A[:, half:]
        return a1 * c - a2 * s, a1 * s + a2 * c

    i1, i2 = rope2(I0)
    p1, p2 = rope2(P0)
    i_ref[:, :half] = i1
    i_ref[:, half:] = i2
    p_ref[:, :half] = p1
    p_ref[:, half:] = p2
    nI = jnp.maximum(jnp.sqrt(jnp.sum(i1 * i1 + i2 * i2, axis=1, keepdims=True)), 1e-12)
    nP = jnp.maximum(jnp.sqrt(jnp.sum(p1 * p1 + p2 * p2, axis=1, keepdims=True)), 1e-12)
    ni_ref[...] = nI
    np_ref[...] = nP
    inv = 1.0 / nP
    pn_ref[:, :half] = p1 * inv
    pn_ref[:, half:] = p2 * inv


def _k1(x2, Wi, Wp, cosb, sinb):
    return pl.pallas_call(
        _proj_body,
        grid=(T // BT,),
        in_specs=[
            pl.BlockSpec((BT, D), lambda i: (i, 0)),
            pl.BlockSpec((D, D), lambda i: (0, 0)),
            pl.BlockSpec((D, D), lambda i: (0, 0)),
            pl.BlockSpec((BT, D // 2), lambda i: (i, 0)),
            pl.BlockSpec((BT, D // 2), lambda i: (i, 0)),
        ],
        out_specs=[
            pl.BlockSpec((BT, D), lambda i: (i, 0)),
            pl.BlockSpec((BT, D), lambda i: (i, 0)),
            pl.BlockSpec((BT, D), lambda i: (i, 0)),
            pl.BlockSpec((BT, 1), lambda i: (i, 0)),
            pl.BlockSpec((BT, 1), lambda i: (i, 0)),
        ],
        out_shape=[
            jax.ShapeDtypeStruct((T, D), jnp.float32),
            jax.ShapeDtypeStruct((T, D), jnp.float32),
            jax.ShapeDtypeStruct((T, D), jnp.float32),
            jax.ShapeDtypeStruct((T, 1), jnp.float32),
            jax.ShapeDtypeStruct((T, 1), jnp.float32),
        ],
    )(x2, Wi, Wp, cosb, sinb)


def _sg_body(i_ref, p_ref, pnt_ref, pns_ref, s_ref, g_ref):
    ti = pl.program_id(0)
    si = pl.program_id(1)
    scale = D ** -0.5
    S = _dotT(i_ref[...], p_ref[...]) * scale
    row = lax.broadcasted_iota(jnp.int32, (BT, BT), 0) + ti * BT
    col = lax.broadcasted_iota(jnp.int32, (BT, BT), 1) + si * BT
    s_ref[...] = jnp.where(row >= col, S, NEG)
    g_ref[...] = _dotT(pnt_ref[...], pns_ref[...])


def _k2(I, P, Pn):
    return pl.pallas_call(
        _sg_body,
        grid=(T // BT, T // BT),
        in_specs=[
            pl.BlockSpec((BT, D), lambda i, j: (i, 0)),
            pl.BlockSpec((BT, D), lambda i, j: (j, 0)),
            pl.BlockSpec((BT, D), lambda i, j: (i, 0)),
            pl.BlockSpec((BT, D), lambda i, j: (j, 0)),
        ],
        out_specs=[
            pl.BlockSpec((BT, BT), lambda i, j: (i, j)),
            pl.BlockSpec((BT, BT), lambda i, j: (i, j)),
        ],
        out_shape=[
            jax.ShapeDtypeStruct((T, T), jnp.float32),
            jax.ShapeDtypeStruct((T, T), jnp.float32),
        ],
    )(I, P, Pn, Pn)


def _topk_body(s_ref, tv_ref, ti_ref, fl_ref):
    Sw = s_ref[...]
    lane = lax.broadcasted_iota(jnp.int32, (BT, T), 1)
    vals = []
    idxs = []
    for _ in range(K):
        m = jnp.max(Sw, axis=1, keepdims=True)
        am = jnp.min(jnp.where(Sw >= m, lane, T), axis=1, keepdims=True)
        vals.append(m)
        idxs.append(am)
        Sw = jnp.where(lane == am, NEG, Sw)
    tv = jnp.concatenate(vals + [jnp.full((BT, 1), NEG, jnp.float32)], axis=1)
    ti = jnp.concatenate(idxs + [jnp.zeros((BT, 1), jnp.int32)], axis=1)
    tv_ref[...] = tv
    ti_ref[...] = ti
    for k in range(K):
        fl_ref[:, k * KS:(k + 1) * KS] = idxs[k] * T + ti


def _k3(S):
    return pl.pallas_call(
        _topk_body,
        grid=(T // BT,),
        in_specs=[pl.BlockSpec((BT, T), lambda i: (i, 0))],
        out_specs=[
            pl.BlockSpec((BT, KS), lambda i: (i, 0)),
            pl.BlockSpec((BT, KS), lambda i: (i, 0)),
            pl.BlockSpec((BT, K * KS), lambda i: (i, 0)),
        ],
        out_shape=[
            jax.ShapeDtypeStruct((T, KS), jnp.float32),
            jax.ShapeDtypeStruct((T, KS), jnp.int32),
            jax.ShapeDtypeStruct((T, K * KS), jnp.int32),
        ],
    )(S)


NW = 32
TPW = T // NW


FPW = TPW * K * KS
IPW = TPW * KS
CH = 128
GRP = 8


def _sc_body(fl_hbm, idx_hbm, png_hbm, np_hbm, g_out, np_out,
             fl_v, g_v, idx_v, np_v, sem1, sem2):
    wid = lax.axis_index("s") * 2 + lax.axis_index("c")
    pltpu.sync_copy(fl_hbm.at[pl.ds(wid * FPW, FPW)], fl_v)
    pltpu.sync_copy(idx_hbm.at[pl.ds(wid * IPW, IPW)], idx_v)

    def gat_g(g, _):
        for b in range(GRP):
            off = (g * GRP + b) * CH
            pltpu.async_copy(png_hbm.at[fl_v.at[pl.ds(off, CH)]],
                             g_v.at[pl.ds(off, CH)], sem1)
        for b in range(GRP):
            off = (g * GRP + b) * CH
            pltpu.make_async_copy(png_hbm.at[fl_v.at[pl.ds(off, CH)]],
                                  g_v.at[pl.ds(off, CH)], sem1).wait()
        return 0

    lax.fori_loop(0, FPW // (CH * GRP), gat_g, 0)

    for b in range(IPW // CH):
        pltpu.async_copy(np_hbm.at[idx_v.at[pl.ds(b * CH, CH)]],
                         np_v.at[pl.ds(b * CH, CH)], sem2)
    for b in range(IPW // CH):
        pltpu.make_async_copy(np_hbm.at[idx_v.at[pl.ds(b * CH, CH)]],
                              np_v.at[pl.ds(b * CH, CH)], sem2).wait()
    pltpu.sync_copy(g_v, g_out.at[pl.ds(wid * FPW, FPW)])
    pltpu.sync_copy(np_v, np_out.at[pl.ds(wid * IPW, IPW)])


def _k4(flat2, idx16, png_flat, np_flat):
    mesh = plsc.VectorSubcoreMesh(core_axis_name="c", subcore_axis_name="s")
    f = functools.partial(
        pl.kernel,
        mesh=mesh,
        out_type=[
            jax.ShapeDtypeStruct((T * K * KS,), jnp.float32),
            jax.ShapeDtypeStruct((T * KS,), jnp.float32),
        ],
        scratch_types=[
            pltpu.VMEM((FPW,), jnp.int32),
            pltpu.VMEM((FPW,), jnp.float32),
            pltpu.VMEM((IPW,), jnp.int32),
            pltpu.VMEM((IPW,), jnp.float32),
            pltpu.SemaphoreType.DMA,
            pltpu.SemaphoreType.DMA,
        ],
    )(_sc_body)
    return f(flat2, idx16, png_flat, np_flat)


BK = 32


def _mega_body(tv_ref, ti_ref, ni_ref, nps_ref, g_ref,
               bw1_ref, br1_ref, bw2_ref, br2_ref, bwh_ref, brh_ref, m_ref,
               y_ref):
    pid = pl.program_id(0)
    tcol = lax.broadcasted_iota(jnp.int32, (BT, 1), 0) + pid * BT
    lane = lax.broadcasted_iota(jnp.int32, (BT, KS), 1)
    keep = (lane <= tcol) & (lane < K)
    kf = keep.astype(jnp.float32)
    tv = tv_ref[...]
    ti = ti_ref[...]
    nI = ni_ref[...]
    nps = nps_ref[...]
    inv_scale = float(D) ** 0.5
    feat = jnp.clip(tv * inv_scale / (nI * nps), -1.0, 1.0) * kf
    delta = jnp.maximum((tcol - ti).astype(jnp.float32), 0.0) * (1.0 / T) * kf
    zpad = jnp.zeros((BT, BK - (K + 2)), jnp.float32)
    parts = []
    for k in range(K):
        gk = jnp.clip(g_ref[:, k * KS:(k + 1) * KS], -1.0, 1.0)
        gmk = gk * kf[:, k:k + 1] * kf
        parts.append(jnp.concatenate(
            [gmk[:, :K], feat[:, k:k + 1], delta[:, k:k + 1], zpad], axis=1))
    rel = jnp.concatenate(parts, axis=1)

    h = _gelu_exact(_dot(rel, bw1_ref[...]) + br1_ref[...])
    h = _gelu_exact(_dot(h, bw2_ref[...]) + br2_ref[...])
    zz = _dot(h, bwh_ref[...]) + brh_ref[...]
    z0 = jnp.tanh(zz[:, :KS])
    z1 = jnp.tanh(zz[:, KS:2 * KS])
    mm = jnp.where(keep, zz[:, 2 * KS:], NEG)
    mx = jnp.max(mm, axis=1, keepdims=True)
    e = jnp.exp(mm - mx) * kf
    w = e / jnp.sum(e, axis=1, keepdims=True)

    ix = jnp.clip((z0 + 1.0) * (0.5 * (PW - 1)), 0.0, PW - 1.0)
    iy = jnp.clip((z1 + 1.0) * (0.5 * (PW - 1)), 0.0, PW - 1.0)
    ix0f = jnp.floor(ix)
    iy0f = jnp.floor(iy)
    wx1 = ix - ix0f
    wy1 = iy - iy0f
    wx0 = 1.0 - wx1
    wy0 = 1.0 - wy1
    ix0 = jnp.clip(ix0f.astype(jnp.int32), 0, PW - 1)
    iy0 = jnp.clip(iy0f.astype(jnp.int32), 0, PW - 1)
    ix1 = jnp.clip(ix0f.astype(jnp.int32) + 1, 0, PW - 1)
    iy1 = jnp.clip(iy0f.astype(jnp.int32) + 1, 0, PW - 1)

    lane256 = lax.broadcasted_iota(jnp.int32, (BT, PW * PW), 1)
    coef = jnp.zeros((BT, PW * PW), jnp.float32)
    for k in range(K):
        wk = w[:, k:k + 1]
        for yy, xx, wy, wx in ((iy0, ix0, wy0, wx0), (iy0, ix1, wy0, wx1),
                               (iy1, ix0, wy1, wx0), (iy1, ix1, wy1, wx1)):
            pos = yy[:, k:k + 1] * PW + xx[:, k:k + 1]
            amp = wk * (wy[:, k:k + 1] * wx[:, k:k + 1])
            coef = coef + jnp.where(lane256 == pos, amp, 0.0)
    y_ref[...] = _dot(coef, m_ref[...])


def _k5p(tv, ti, nI, nps, G, BW1, br1, BW2, br2, BWh, brh, M):
    return pl.pallas_call(
        _mega_body,
        grid=(T // BT,),
        in_specs=[
            pl.BlockSpec((BT, KS), lambda i: (i, 0)),
            pl.BlockSpec((BT, KS), lambda i: (i, 0)),
            pl.BlockSpec((BT, 1), lambda i: (i, 0)),
            pl.BlockSpec((BT, KS), lambda i: (i, 0)),
            pl.BlockSpec((BT, K * KS), lambda i: (i, 0)),
            pl.BlockSpec((K * BK, K * RH), lambda i: (0, 0)),
            pl.BlockSpec((1, K * RH), lambda i: (0, 0)),
            pl.BlockSpec((K * RH, K * RH), lambda i: (0, 0)),
            pl.BlockSpec((1, K * RH), lambda i: (0, 0)),
            pl.BlockSpec((K * RH, 3 * KS), lambda i: (0, 0)),
            pl.BlockSpec((1, 3 * KS), lambda i: (0, 0)),
            pl.BlockSpec((PW * PW, D), lambda i: (0, 0)),
        ],
        out_specs=pl.BlockSpec((BT, D), lambda i: (i, 0)),
        out_shape=jax.ShapeDtypeStruct((T, D), jnp.float32),
    )(tv, ti, nI, nps, G, BW1, br1, BW2, br2, BWh, brh, M)


def _gelu_exact(x):
    return x * 0.5 * (1.0 + lax.erf(x * (2.0 ** -0.5)))


def _pal_body(pal_ref, wo_ref, m_ref):
    m_ref[...] = lax.dot_general(pal_ref[...].astype(jnp.bfloat16),
                                 wo_ref[...].astype(jnp.bfloat16),
                                 (((0,), (0,)), ((), ())),
                                 preferred_element_type=jnp.float32)


def _k8(pal2, Wo0):
    return pl.pallas_call(
        _pal_body,
        in_specs=[
            pl.BlockSpec((D, PW * PW), lambda: (0, 0)),
            pl.BlockSpec((D, D), lambda: (0, 0)),
        ],
        out_specs=pl.BlockSpec((PW * PW, D), lambda: (0, 0)),
        out_shape=jax.ShapeDtypeStruct((PW * PW, D), jnp.float32),
    )(pal2, Wo0)


def kernel(x, Wi, Wp, palette, W1, b1, W2, b2, Wc, bc, Wm, bm, Wo):
    x2 = x.reshape(T, D)
    half = D // 2
    freqs = 1.0 / (10000.0 ** (jnp.arange(half, dtype=jnp.float32) / half))
    ang = jnp.arange(T, dtype=jnp.float32)[:, None] * freqs[None, :]
    cosb = jnp.cos(ang)
    sinb = jnp.sin(ang)
    I, P, Pn, nI, nP = _k1(x2, Wi, Wp, cosb, sinb)
    S, PnG = _k2(I, P, Pn)
    tv, ti, flat = _k3(S)

    flat2 = flat.reshape(T * K * KS)
    idx1 = ti.reshape(T * KS)
    png_flat = PnG.reshape(T * T)
    np_flat = nP.reshape(T)
    g_flat, nps1 = _k4(flat2, idx1, png_flat, np_flat)
    G = g_flat.reshape(T, K * KS)
    nps = nps1.reshape(T, KS)

    eye = jnp.eye(K, dtype=jnp.float32)
    W1pad = jnp.pad(W1.T, ((0, BK - (K + 2)), (0, 0)))
    BW1 = (eye[:, None, :, None] * W1pad[None, :, None, :]).reshape(K * BK, K * RH)
    BW2 = (eye[:, None, :, None] * W2.T[None, :, None, :]).reshape(K * RH, K * RH)
    Whc = jnp.concatenate([Wc.T, Wm.T], axis=1)
    BWh = (eye[:, None, None, :] * Whc[None, :, :, None]).reshape(K * RH, 3 * K)
    BWh = BWh.reshape(K * RH, 3, K)
    BWh = jnp.pad(BWh, ((0, 0), (0, 0), (0, KS - K))).reshape(K * RH, 3 * KS)
    br1 = jnp.broadcast_to(b1[None, :], (K, RH)).reshape(1, K * RH)
    br2 = jnp.broadcast_to(b2[None, :], (K, RH)).reshape(1, K * RH)
    bh3 = jnp.concatenate([bc, bm])
    brh = jnp.broadcast_to(bh3[:, None], (3, KS)).reshape(1, 3 * KS)

    pal2 = palette.reshape(D, PW * PW)
    M = _k8(pal2, Wo[0])
    y = _k5p(tv, ti, nI, nps, G, BW1, br1, BW2, br2, BWh, brh, M)
    return y.reshape(1, T, D)


# speedup vs baseline: 1.4224x; 1.0480x over previous
import functools

import jax
import jax.numpy as jnp
from jax import lax
from jax.experimental import pallas as pl
from jax.experimental.pallas import tpu as pltpu
from jax.experimental.pallas import tpu_sc as plsc

T = 2048
D = 1024
K = 15
KS = 16
PW = 16
RH = 64
BT = 256
NEG = -1e30
HP = jax.lax.Precision.HIGHEST


def _dotT(a, b):
    return lax.dot_general(a.astype(jnp.bfloat16), b.astype(jnp.bfloat16),
                           (((1,), (1,)), ((), ())),
                           preferred_element_type=jnp.float32)


def _dot(a, b):
    return lax.dot_general(a.astype(jnp.bfloat16), b.astype(jnp.bfloat16),
                           (((1,), (0,)), ((), ())),
                           preferred_element_type=jnp.float32)


def _proj_body(x_ref, wi_ref, wp_ref, c_ref, s_ref, i_ref, p_ref, pn_ref, ni_ref, np_ref):
    half = D // 2
    x = x_ref[...]
    I0 = _dotT(x, wi_ref[...])
    P0 = _dotT(x, wp_ref[...])
    c = c_ref[...]
    s = s_ref[...]

    def rope2(A):
        a1 = A[:, :half]
        a2 = A[:, half:]
        return a1 * c - a2 * s, a1 * s + a2 * c

    i1, i2 = rope2(I0)
    p1, p2 = rope2(P0)
    i_ref[:, :half] = i1
    i_ref[:, half:] = i2
    p_ref[:, :half] = p1
    p_ref[:, half:] = p2
    nI = jnp.maximum(jnp.sqrt(jnp.sum(i1 * i1 + i2 * i2, axis=1, keepdims=True)), 1e-12)
    nP = jnp.maximum(jnp.sqrt(jnp.sum(p1 * p1 + p2 * p2, axis=1, keepdims=True)), 1e-12)
    ni_ref[...] = nI
    np_ref[...] = nP
    inv = 1.0 / nP
    pn_ref[:, :half] = p1 * inv
    pn_ref[:, half:] = p2 * inv


def _k1(x2, Wi, Wp, cosb, sinb):
    return pl.pallas_call(
        _proj_body,
        grid=(T // BT,),
        in_specs=[
            pl.BlockSpec((BT, D), lambda i: (i, 0)),
            pl.BlockSpec((D, D), lambda i: (0, 0)),
            pl.BlockSpec((D, D), lambda i: (0, 0)),
            pl.BlockSpec((BT, D // 2), lambda i: (i, 0)),
            pl.BlockSpec((BT, D // 2), lambda i: (i, 0)),
        ],
        out_specs=[
            pl.BlockSpec((BT, D), lambda i: (i, 0)),
            pl.BlockSpec((BT, D), lambda i: (i, 0)),
            pl.BlockSpec((BT, D), lambda i: (i, 0)),
            pl.BlockSpec((BT, 1), lambda i: (i, 0)),
            pl.BlockSpec((BT, 1), lambda i: (i, 0)),
        ],
        out_shape=[
            jax.ShapeDtypeStruct((T, D), jnp.float32),
            jax.ShapeDtypeStruct((T, D), jnp.float32),
            jax.ShapeDtypeStruct((T, D), jnp.float32),
            jax.ShapeDtypeStruct((T, 1), jnp.float32),
            jax.ShapeDtypeStruct((T, 1), jnp.float32),
        ],
    )(x2, Wi, Wp, cosb, sinb)


def _sg_body(i_ref, p_ref, pnt_ref, pns_ref, s_ref, g_ref):
    ti = pl.program_id(0)
    si = pl.program_id(1)
    scale = D ** -0.5
    S = _dotT(i_ref[...], p_ref[...]) * scale
    row = lax.broadcasted_iota(jnp.int32, (BT, BT), 0) + ti * BT
    col = lax.broadcasted_iota(jnp.int32, (BT, BT), 1) + si * BT
    s_ref[...] = jnp.where(row >= col, S, NEG)
    g_ref[...] = _dotT(pnt_ref[...], pns_ref[...])


def _k2(I, P, Pn):
    return pl.pallas_call(
        _sg_body,
        grid=(T // BT, T // BT),
        in_specs=[
            pl.BlockSpec((BT, D), lambda i, j: (i, 0)),
            pl.BlockSpec((BT, D), lambda i, j: (j, 0)),
            pl.BlockSpec((BT, D), lambda i, j: (i, 0)),
            pl.BlockSpec((BT, D), lambda i, j: (j, 0)),
        ],
        out_specs=[
            pl.BlockSpec((BT, BT), lambda i, j: (i, j)),
            pl.BlockSpec((BT, BT), lambda i, j: (i, j)),
        ],
        out_shape=[
            jax.ShapeDtypeStruct((T, T), jnp.float32),
            jax.ShapeDtypeStruct((T, T), jnp.float32),
        ],
    )(I, P, Pn, Pn)


def _topk_body(s_ref, tv_ref, ti_ref, fl_ref):
    Sw = s_ref[...]
    W = Sw.shape[1]
    lane = lax.broadcasted_iota(jnp.int32, (BT, W), 1)
    vals = []
    idxs = []
    for _ in range(K):
        m = jnp.max(Sw, axis=1, keepdims=True)
        am = jnp.min(jnp.where(Sw >= m, lane, T), axis=1, keepdims=True)
        vals.append(m)
        idxs.append(am)
        Sw = jnp.where(lane == am, NEG, Sw)
    tv = jnp.concatenate(vals + [jnp.full((BT, 1), NEG, jnp.float32)], axis=1)
    ti = jnp.concatenate(idxs + [jnp.zeros((BT, 1), jnp.int32)], axis=1)
    tv_ref[...] = tv
    ti_ref[...] = ti
    for k in range(K):
        fl_ref[:, k * KS:(k + 1) * KS] = idxs[k] * T + ti


def _k3(S, W, row0, nrow):
    rb = row0 // BT
    return pl.pallas_call(
        _topk_body,
        grid=(nrow // BT,),
        in_specs=[pl.BlockSpec((BT, W), lambda i: (i + rb, 0))],
        out_specs=[
            pl.BlockSpec((BT, KS), lambda i: (i, 0)),
            pl.BlockSpec((BT, KS), lambda i: (i, 0)),
            pl.BlockSpec((BT, K * KS), lambda i: (i, 0)),
        ],
        out_shape=[
            jax.ShapeDtypeStruct((nrow, KS), jnp.float32),
            jax.ShapeDtypeStruct((nrow, KS), jnp.int32),
            jax.ShapeDtypeStruct((nrow, K * KS), jnp.int32),
        ],
    )(S)


NW = 32
TPW = T // NW


FPW = TPW * K * KS
IPW = TPW * KS
CH = 512
GRP = 6


def _sc_body(fl_hbm, idx_hbm, png_hbm, np_hbm, g_out, np_out,
             fl_v, g_v, idx_v, np_v, sem1, sem2):
    wid = lax.axis_index("s") * 2 + lax.axis_index("c")
    pltpu.sync_copy(fl_hbm.at[pl.ds(wid * FPW, FPW)], fl_v)
    pltpu.sync_copy(idx_hbm.at[pl.ds(wid * IPW, IPW)], idx_v)

    def gat_g(g, _):
        for b in range(GRP):
            off = (g * GRP + b) * CH
            pltpu.async_copy(png_hbm.at[fl_v.at[pl.ds(off, CH)]],
                             g_v.at[pl.ds(off, CH)], sem1)
        for b in range(GRP):
            off = (g * GRP + b) * CH
            pltpu.make_async_copy(png_hbm.at[fl_v.at[pl.ds(off, CH)]],
                                  g_v.at[pl.ds(off, CH)], sem1).wait()
        return 0

    lax.fori_loop(0, FPW // (CH * GRP), gat_g, 0)

    for b in range(IPW // CH):
        pltpu.async_copy(np_hbm.at[idx_v.at[pl.ds(b * CH, CH)]],
                         np_v.at[pl.ds(b * CH, CH)], sem2)
    for b in range(IPW // CH):
        pltpu.make_async_copy(np_hbm.at[idx_v.at[pl.ds(b * CH, CH)]],
                              np_v.at[pl.ds(b * CH, CH)], sem2).wait()
    pltpu.sync_copy(g_v, g_out.at[pl.ds(wid * FPW, FPW)])
    pltpu.sync_copy(np_v, np_out.at[pl.ds(wid * IPW, IPW)])


def _k4(flat2, idx16, png_flat, np_flat):
    mesh = plsc.VectorSubcoreMesh(core_axis_name="c", subcore_axis_name="s")
    f = functools.partial(
        pl.kernel,
        mesh=mesh,
        out_type=[
            jax.ShapeDtypeStruct((T * K * KS,), jnp.float32),
            jax.ShapeDtypeStruct((T * KS,), jnp.float32),
        ],
        scratch_types=[
            pltpu.VMEM((FPW,), jnp.int32),
            pltpu.VMEM((FPW,), jnp.float32),
            pltpu.VMEM((IPW,), jnp.int32),
            pltpu.VMEM((IPW,), jnp.float32),
            pltpu.SemaphoreType.DMA,
            pltpu.SemaphoreType.DMA,
        ],
    )(_sc_body)
    return f(flat2, idx16, png_flat, np_flat)


BK = 32


def _mega_body(tv_ref, ti_ref, ni_ref, nps_ref, g_ref,
               bw1_ref, br1_ref, bw2_ref, br2_ref, bwh_ref, brh_ref, m_ref,
               y_ref):
    pid = pl.program_id(0)
    tcol = lax.broadcasted_iota(jnp.int32, (BT, 1), 0) + pid * BT
    lane = lax.broadcasted_iota(jnp.int32, (BT, KS), 1)
    keep = (lane <= tcol) & (lane < K)
    kf = keep.astype(jnp.float32)
    tv = tv_ref[...]
    ti = ti_ref[...]
    nI = ni_ref[...]
    nps = nps_ref[...]
    inv_scale = float(D) ** 0.5
    feat = jnp.clip(tv * inv_scale / (nI * nps), -1.0, 1.0) * kf
    delta = jnp.maximum((tcol - ti).astype(jnp.float32), 0.0) * (1.0 / T) * kf
    zpad = jnp.zeros((BT, BK - (K + 2)), jnp.float32)
    parts = []
    for k in range(K):
        gk = jnp.clip(g_ref[:, k * KS:(k + 1) * KS], -1.0, 1.0)
        gmk = gk * kf[:, k:k + 1] * kf
        parts.append(jnp.concatenate(
            [gmk[:, :K], feat[:, k:k + 1], delta[:, k:k + 1], zpad], axis=1))
    rel = jnp.concatenate(parts, axis=1)

    h = _gelu_exact(_dot(rel, bw1_ref[...]) + br1_ref[...])
    h = _gelu_exact(_dot(h, bw2_ref[...]) + br2_ref[...])
    zz = _dot(h, bwh_ref[...]) + brh_ref[...]
    z0 = jnp.tanh(zz[:, :KS])
    z1 = jnp.tanh(zz[:, KS:2 * KS])
    mm = jnp.where(keep, zz[:, 2 * KS:], NEG)
    mx = jnp.max(mm, axis=1, keepdims=True)
    e = jnp.exp(mm - mx) * kf
    w = e / jnp.sum(e, axis=1, keepdims=True)

    ix = jnp.clip((z0 + 1.0) * (0.5 * (PW - 1)), 0.0, PW - 1.0)
    iy = jnp.clip((z1 + 1.0) * (0.5 * (PW - 1)), 0.0, PW - 1.0)
    ix0f = jnp.floor(ix)
    iy0f = jnp.floor(iy)
    wx1 = ix - ix0f
    wy1 = iy - iy0f
    wx0 = 1.0 - wx1
    wy0 = 1.0 - wy1
    ix0 = jnp.clip(ix0f.astype(jnp.int32), 0, PW - 1)
    iy0 = jnp.clip(iy0f.astype(jnp.int32), 0, PW - 1)
    ix1 = jnp.clip(ix0f.astype(jnp.int32) + 1, 0, PW - 1)
    iy1 = jnp.clip(iy0f.astype(jnp.int32) + 1, 0, PW - 1)

    lane256 = lax.broadcasted_iota(jnp.int32, (BT, PW * PW), 1)
    coef = jnp.zeros((BT, PW * PW), jnp.float32)
    for k in range(K):
        wk = w[:, k:k + 1]
        for yy, xx, wy, wx in ((iy0, ix0, wy0, wx0), (iy0, ix1, wy0, wx1),
                               (iy1, ix0, wy1, wx0), (iy1, ix1, wy1, wx1)):
            pos = yy[:, k:k + 1] * PW + xx[:, k:k + 1]
            amp = wk * (wy[:, k:k + 1] * wx[:, k:k + 1])
            coef = coef + jnp.where(lane256 == pos, amp, 0.0)
    y_ref[...] = _dot(coef, m_ref[...])


def _k5p(tv, ti, nI, nps, G, BW1, br1, BW2, br2, BWh, brh, M):
    return pl.pallas_call(
        _mega_body,
        grid=(T // BT,),
        in_specs=[
            pl.BlockSpec((BT, KS), lambda i: (i, 0)),
            pl.BlockSpec((BT, KS), lambda i: (i, 0)),
            pl.BlockSpec((BT, 1), lambda i: (i, 0)),
            pl.BlockSpec((BT, KS), lambda i: (i, 0)),
            pl.BlockSpec((BT, K * KS), lambda i: (i, 0)),
            pl.BlockSpec((K * BK, K * RH), lambda i: (0, 0)),
            pl.BlockSpec((1, K * RH), lambda i: (0, 0)),
            pl.BlockSpec((K * RH, K * RH), lambda i: (0, 0)),
            pl.BlockSpec((1, K * RH), lambda i: (0, 0)),
            pl.BlockSpec((K * RH, 3 * KS), lambda i: (0, 0)),
            pl.BlockSpec((1, 3 * KS), lambda i: (0, 0)),
            pl.BlockSpec((PW * PW, D), lambda i: (0, 0)),
        ],
        out_specs=pl.BlockSpec((BT, D), lambda i: (i, 0)),
        out_shape=jax.ShapeDtypeStruct((T, D), jnp.float32),
    )(tv, ti, nI, nps, G, BW1, br1, BW2, br2, BWh, brh, M)


def _gelu_exact(x):
    return x * 0.5 * (1.0 + lax.erf(x * (2.0 ** -0.5)))


def _pal_body(pal_ref, wo_ref, m_ref):
    m_ref[...] = lax.dot_general(pal_ref[...].astype(jnp.bfloat16),
                                 wo_ref[...].astype(jnp.bfloat16),
                                 (((0,), (0,)), ((), ())),
                                 preferred_element_type=jnp.float32)


def _k8(pal2, Wo0):
    return pl.pallas_call(
        _pal_body,
        in_specs=[
            pl.BlockSpec((D, PW * PW), lambda: (0, 0)),
            pl.BlockSpec((D, D), lambda: (0, 0)),
        ],
        out_specs=pl.BlockSpec((PW * PW, D), lambda: (0, 0)),
        out_shape=jax.ShapeDtypeStruct((PW * PW, D), jnp.float32),
    )(pal2, Wo0)


def kernel(x, Wi, Wp, palette, W1, b1, W2, b2, Wc, bc, Wm, bm, Wo):
    x2 = x.reshape(T, D)
    half = D // 2
    freqs = 1.0 / (10000.0 ** (jnp.arange(half, dtype=jnp.float32) / half))
    ang = jnp.arange(T, dtype=jnp.float32)[:, None] * freqs[None, :]
    cosb = jnp.cos(ang)
    sinb = jnp.sin(ang)
    I, P, Pn, nI, nP = _k1(x2, Wi, Wp, cosb, sinb)
    S, PnG = _k2(I, P, Pn)
    tvA, tiA, flA = _k3(S, T // 2, 0, T // 2)
    tvB, tiB, flB = _k3(S, T, T // 2, T // 2)
    tv = jnp.concatenate([tvA, tvB], axis=0)
    ti = jnp.concatenate([tiA, tiB], axis=0)
    flat = jnp.concatenate([flA, flB], axis=0)

    flat2 = flat.reshape(T * K * KS)
    idx1 = ti.reshape(T * KS)
    png_flat = PnG.reshape(T * T)
    np_flat = nP.reshape(T)
    g_flat, nps1 = _k4(flat2, idx1, png_flat, np_flat)
    G = g_flat.reshape(T, K * KS)
    nps = nps1.reshape(T, KS)

    eye = jnp.eye(K, dtype=jnp.float32)
    W1pad = jnp.pad(W1.T, ((0, BK - (K + 2)), (0, 0)))
    BW1 = (eye[:, None, :, None] * W1pad[None, :, None, :]).reshape(K * BK, K * RH)
    BW2 = (eye[:, None, :, None] * W2.T[None, :, None, :]).reshape(K * RH, K * RH)
    Whc = jnp.concatenate([Wc.T, Wm.T], axis=1)
    BWh = (eye[:, None, None, :] * Whc[None, :, :, None]).reshape(K * RH, 3 * K)
    BWh = BWh.reshape(K * RH, 3, K)
    BWh = jnp.pad(BWh, ((0, 0), (0, 0), (0, KS - K))).reshape(K * RH, 3 * KS)
    br1 = jnp.broadcast_to(b1[None, :], (K, RH)).reshape(1, K * RH)
    br2 = jnp.broadcast_to(b2[None, :], (K, RH)).reshape(1, K * RH)
    bh3 = jnp.concatenate([bc, bm])
    brh = jnp.broadcast_to(bh3[:, None], (3, KS)).reshape(1, 3 * KS)

    pal2 = palette.reshape(D, PW * PW)
    M = _k8(pal2, Wo[0])
    y = _k5p(tv, ti, nI, nps, G, BW1, br1, BW2, br2, BWh, brh, M)
    return y.reshape(1, T, D)
